# Initial kernel scaffold; baseline (speedup 1.0000x reference)
import jax
import jax.numpy as jnp
from jax.experimental import pallas as pl


def kernel(idx, table):
    raise NotImplementedError("write your pallas kernel here")


# speedup vs baseline: 1.0399x; 1.0399x over previous
import functools

import jax
import jax.numpy as jnp
from jax import lax
from jax.experimental import pallas as pl
from jax.experimental.pallas import tpu as pltpu
from jax.experimental.pallas import tpu_sc as plsc

VOCAB = 1000
NC, NS = 2, 16
NW = NC * NS
B_TOTAL = 1024 * 50
ROWS_PER_W = B_TOTAL // NW
CHUNK = 40
NBUF = 2
NCHUNKS = ROWS_PER_W // CHUNK


def _gather_body(table_hbm, idx_hbm, out_hbm, idx_v, bufs, gsems, ssems):
    wid = lax.axis_index("s") * NC + lax.axis_index("c")
    base = wid * ROWS_PER_W

    pltpu.sync_copy(idx_hbm.at[pl.ds(base, ROWS_PER_W)], idx_v)

    def start_gather(j, b):
        pltpu.async_copy(
            table_hbm.at[idx_v.at[pl.ds(j * CHUNK, CHUNK)]], bufs[b], gsems[b]
        )

    def start_store(j, b):
        pltpu.async_copy(
            bufs[b], out_hbm.at[pl.ds(base + j * CHUNK, CHUNK)], ssems[b]
        )

    for b in range(NBUF):
        start_gather(b, b)

    @pl.loop(0, NCHUNKS, step=NBUF)
    def _(g):
        for b in range(NBUF):
            j = g + b
            pltpu.make_async_copy(
                table_hbm.at[idx_v.at[pl.ds(0, CHUNK)]], bufs[b], gsems[b]
            ).wait()
            start_store(j, b)
            @pl.when(j + NBUF < NCHUNKS)
            def _():
                pltpu.make_async_copy(
                    bufs[b], out_hbm.at[pl.ds(base, CHUNK)], ssems[b]
                ).wait()
                start_gather(j + NBUF, b)

    for b in range(NBUF):
        pltpu.make_async_copy(
            bufs[b], out_hbm.at[pl.ds(base, CHUNK)], ssems[b]
        ).wait()


@jax.jit
def _lookup(idx_flat, table):
    mesh = plsc.VectorSubcoreMesh(core_axis_name="c", subcore_axis_name="s")
    run = pl.kernel(
        _gather_body,
        out_type=jax.ShapeDtypeStruct((B_TOTAL, VOCAB), jnp.float32),
        mesh=mesh,
        compiler_params=pltpu.CompilerParams(use_tc_tiling_on_sc=False),
        scratch_types=[
            pltpu.VMEM((ROWS_PER_W,), jnp.int32),
            [pltpu.VMEM((CHUNK, VOCAB), jnp.float32) for _ in range(NBUF)],
            [pltpu.SemaphoreType.DMA for _ in range(NBUF)],
            [pltpu.SemaphoreType.DMA for _ in range(NBUF)],
        ],
    )
    return run(table, idx_flat)


def kernel(idx, table):
    b, t = idx.shape
    out = _lookup(idx.reshape(-1).astype(jnp.int32), table)
    return out.reshape(b, t, VOCAB)
